# baseline (speedup 1.0000x reference)
import functools
import jax
import jax.numpy as jnp
from jax import lax
from jax.experimental import pallas as pl
from jax.experimental.pallas import tpu as pltpu
from jax.experimental.pallas import tpu_sc as plsc

EPS = 1e-6

N, C, D, H, W = 2, 4, 128, 128, 128
ROWS = N * C
ROW_ELEMS = D * H * W
SLAB = H * W

D_TC = 0
D_SC = D - D_TC

BLK_D = 32
ND = D_TC // BLK_D if D_TC else 0


def _tc_kernel(inp_ref, tgt_ref, out_pt, out_den):
    n = pl.program_id(0)
    c = pl.program_id(1)
    d = pl.program_id(2)
    row = n * C + c

    @pl.when(jnp.logical_and(row == 0, d == 0))
    def _init():
        out_pt[...] = jnp.zeros_like(out_pt)
        out_den[...] = jnp.zeros_like(out_den)

    x = inp_ref[0, 0]
    t = tgt_ref[0, 0]
    p = jax.nn.sigmoid(x)
    s_pt = jnp.sum(p * t)
    s_den = jnp.sum(p * p + t)

    row_mask = jax.lax.broadcasted_iota(jnp.int32, (ROWS, 128), 0) == row
    out_pt[...] += jnp.where(row_mask, s_pt, 0.0)
    out_den[...] += jnp.where(row_mask, s_den, 0.0)


def _tc_partials(input, target):
    return pl.pallas_call(
        _tc_kernel,
        grid=(N, C, ND),
        in_specs=[
            pl.BlockSpec((1, 1, BLK_D, H, W), lambda n, c, d: (n, c, d, 0, 0)),
            pl.BlockSpec((1, 1, BLK_D, H, W), lambda n, c, d: (n, c, d, 0, 0)),
        ],
        out_specs=[
            pl.BlockSpec((ROWS, 128), lambda n, c, d: (0, 0)),
            pl.BlockSpec((ROWS, 128), lambda n, c, d: (0, 0)),
        ],
        out_shape=[
            jax.ShapeDtypeStruct((ROWS, 128), jnp.float32),
            jax.ShapeDtypeStruct((ROWS, 128), jnp.float32),
        ],
    )(input, target)


CHUNK = 16384
NW = 32
SC_ROW = D_SC * SLAB
PER_W = SC_ROW // 4
NCH = PER_W // CHUNK if D_SC else 0

_sc_mesh = plsc.VectorSubcoreMesh(core_axis_name="c", subcore_axis_name="s")


@functools.partial(
    pl.kernel,
    out_type=jax.ShapeDtypeStruct((NW, 2, 16), jnp.float32),
    mesh=_sc_mesh,
    scratch_types=[
        pltpu.VMEM((CHUNK,), jnp.float32),
        pltpu.VMEM((CHUNK,), jnp.float32),
        pltpu.VMEM((2, 16), jnp.float32),
    ],
)
def _sc_partials(inp_hbm, tgt_hbm, out_hbm, xbuf, tbuf, res):
    w = lax.axis_index("s") * 2 + lax.axis_index("c")
    row = w // 4
    q = w % 4
    base = row * ROW_ELEMS + D_TC * SLAB + q * PER_W

    def chunk_body(g, accs):
        a_pt, a_den = accs
        off = pl.multiple_of(base + g * CHUNK, CHUNK)
        pltpu.sync_copy(inp_hbm.at[pl.ds(off, CHUNK)], xbuf)
        pltpu.sync_copy(tgt_hbm.at[pl.ds(off, CHUNK)], tbuf)

        def inner(k, accs2):
            b_pt, b_den = accs2
            o = pl.multiple_of(k * 16, 16)
            x = xbuf[pl.ds(o, 16)]
            t = tbuf[pl.ds(o, 16)]
            p = 1.0 / (1.0 + jnp.exp(-x))
            return (b_pt + p * t, b_den + (p * p + t))

        return lax.fori_loop(0, CHUNK // 16, inner, (a_pt, a_den))

    z = jnp.zeros((16,), jnp.float32)
    a_pt, a_den = lax.fori_loop(0, NCH, chunk_body, (z, z))
    res[0, :] = a_pt
    res[1, :] = a_den
    pltpu.sync_copy(res, out_hbm.at[w])


def kernel(input, target):
    s_pt = jnp.zeros((ROWS,), jnp.float32)
    s_den = jnp.zeros((ROWS,), jnp.float32)

    if D_TC:
        tc_pt, tc_den = _tc_partials(input, target)
        s_pt = s_pt + tc_pt[:, 0]
        s_den = s_den + tc_den[:, 0]

    if D_SC:
        part = _sc_partials(input.reshape(-1), target.reshape(-1))
        s_pt = s_pt + part[:, 0, :].reshape(ROWS, 4 * 16).sum(-1)
        s_den = s_den + part[:, 1, :].reshape(ROWS, 4 * 16).sum(-1)

    intersect = s_pt[0:C] + s_pt[C:ROWS]
    denom = s_den[0:C] + s_den[C:ROWS]
    dice = 2.0 * intersect / jnp.maximum(denom, EPS)
    loss = 1.0 - jnp.mean(dice)
    return (loss, dice)


# speedup vs baseline: 3.3703x; 3.3703x over previous
import functools
import jax
import jax.numpy as jnp
from jax import lax
from jax.experimental import pallas as pl
from jax.experimental.pallas import tpu as pltpu
from jax.experimental.pallas import tpu_sc as plsc

EPS = 1e-6

N, C, D, H, W = 2, 4, 128, 128, 128
ROWS = N * C
ROW_ELEMS = D * H * W
SLAB = H * W

D_TC = 96
D_SC = D - D_TC

BLK_D = 32
ND = D_TC // BLK_D if D_TC else 0


def _tc_kernel(inp_ref, tgt_ref, out_pt, out_den):
    n = pl.program_id(0)
    c = pl.program_id(1)
    d = pl.program_id(2)
    row = n * C + c

    @pl.when(jnp.logical_and(row == 0, d == 0))
    def _init():
        out_pt[...] = jnp.zeros_like(out_pt)
        out_den[...] = jnp.zeros_like(out_den)

    x = inp_ref[0, 0]
    t = tgt_ref[0, 0]
    p = jax.nn.sigmoid(x)
    s_pt = jnp.sum(p * t)
    s_den = jnp.sum(p * p + t)

    row_mask = jax.lax.broadcasted_iota(jnp.int32, (ROWS, 128), 0) == row
    out_pt[...] += jnp.where(row_mask, s_pt, 0.0)
    out_den[...] += jnp.where(row_mask, s_den, 0.0)


def _tc_partials(input, target):
    return pl.pallas_call(
        _tc_kernel,
        grid=(N, C, ND),
        in_specs=[
            pl.BlockSpec((1, 1, BLK_D, H, W), lambda n, c, d: (n, c, d, 0, 0)),
            pl.BlockSpec((1, 1, BLK_D, H, W), lambda n, c, d: (n, c, d, 0, 0)),
        ],
        out_specs=[
            pl.BlockSpec((ROWS, 128), lambda n, c, d: (0, 0)),
            pl.BlockSpec((ROWS, 128), lambda n, c, d: (0, 0)),
        ],
        out_shape=[
            jax.ShapeDtypeStruct((ROWS, 128), jnp.float32),
            jax.ShapeDtypeStruct((ROWS, 128), jnp.float32),
        ],
    )(input, target)


CHUNK = 16384
NW = 32
SC_ROW = D_SC * SLAB
PER_W = SC_ROW // 4
NCH = PER_W // CHUNK if D_SC else 0

_sc_mesh = plsc.VectorSubcoreMesh(core_axis_name="c", subcore_axis_name="s")


@functools.partial(
    pl.kernel,
    out_type=jax.ShapeDtypeStruct((NW, 2, 16), jnp.float32),
    mesh=_sc_mesh,
    scratch_types=[
        pltpu.VMEM((CHUNK,), jnp.float32),
        pltpu.VMEM((CHUNK,), jnp.float32),
        pltpu.VMEM((CHUNK,), jnp.float32),
        pltpu.VMEM((CHUNK,), jnp.float32),
        pltpu.VMEM((2, 16), jnp.float32),
        pltpu.SemaphoreType.DMA,
        pltpu.SemaphoreType.DMA,
        pltpu.SemaphoreType.DMA,
        pltpu.SemaphoreType.DMA,
    ],
)
def _sc_partials(inp_hbm, tgt_hbm, out_hbm,
                 xbuf0, xbuf1, tbuf0, tbuf1, res,
                 semx0, semx1, semt0, semt1):
    w = lax.axis_index("s") * 2 + lax.axis_index("c")
    row = w // 4
    q = w % 4
    base = row * ROW_ELEMS + D_TC * SLAB + q * PER_W

    def start(g, xbuf, tbuf, semx, semt):
        off = pl.multiple_of(base + g * CHUNK, CHUNK)
        pltpu.async_copy(inp_hbm.at[pl.ds(off, CHUNK)], xbuf, semx)
        pltpu.async_copy(tgt_hbm.at[pl.ds(off, CHUNK)], tbuf, semt)

    def wait(xbuf, tbuf, semx, semt):
        pltpu.make_async_copy(inp_hbm.at[pl.ds(0, CHUNK)], xbuf, semx).wait()
        pltpu.make_async_copy(tgt_hbm.at[pl.ds(0, CHUNK)], tbuf, semt).wait()

    def compute(xbuf, tbuf, accs):
        def inner(k, accs2):
            o = pl.multiple_of(k * 64, 64)
            out = []
            for j in range(4):
                a_pt, a_den = accs2[2 * j], accs2[2 * j + 1]
                x = xbuf[pl.ds(o + j * 16, 16)]
                t = tbuf[pl.ds(o + j * 16, 16)]
                p = 1.0 / (1.0 + jnp.exp(-x))
                out.append(a_pt + p * t)
                out.append(a_den + (p * p + t))
            return tuple(out)

        return lax.fori_loop(0, CHUNK // 64, inner, accs, unroll=4)

    start(0, xbuf0, tbuf0, semx0, semt0)
    z = jnp.zeros((16,), jnp.float32)
    accs = (z,) * 8

    def two_chunks(gp, accs):
        g0 = gp * 2
        start(g0 + 1, xbuf1, tbuf1, semx1, semt1)
        wait(xbuf0, tbuf0, semx0, semt0)
        accs = compute(xbuf0, tbuf0, accs)

        @pl.when(g0 + 2 < NCH)
        def _():
            start(g0 + 2, xbuf0, tbuf0, semx0, semt0)

        wait(xbuf1, tbuf1, semx1, semt1)
        return compute(xbuf1, tbuf1, accs)

    accs = lax.fori_loop(0, NCH // 2, two_chunks, accs)
    a_pt = (accs[0] + accs[2]) + (accs[4] + accs[6])
    a_den = (accs[1] + accs[3]) + (accs[5] + accs[7])
    res[0, :] = a_pt
    res[1, :] = a_den
    pltpu.sync_copy(res, out_hbm.at[w])


def kernel(input, target):
    s_pt = jnp.zeros((ROWS,), jnp.float32)
    s_den = jnp.zeros((ROWS,), jnp.float32)

    if D_TC:
        tc_pt, tc_den = _tc_partials(input, target)
        s_pt = s_pt + tc_pt[:, 0]
        s_den = s_den + tc_den[:, 0]

    if D_SC:
        part = _sc_partials(input.reshape(-1), target.reshape(-1))
        s_pt = s_pt + part[:, 0, :].reshape(ROWS, 4 * 16).sum(-1)
        s_den = s_den + part[:, 1, :].reshape(ROWS, 4 * 16).sum(-1)

    intersect = s_pt[0:C] + s_pt[C:ROWS]
    denom = s_den[0:C] + s_den[C:ROWS]
    dice = 2.0 * intersect / jnp.maximum(denom, EPS)
    loss = 1.0 - jnp.mean(dice)
    return (loss, dice)


# speedup vs baseline: 3.5456x; 1.0520x over previous
import functools
import jax
import jax.numpy as jnp
from jax import lax
from jax.experimental import pallas as pl
from jax.experimental.pallas import tpu as pltpu
from jax.experimental.pallas import tpu_sc as plsc

EPS = 1e-6

N, C, D, H, W = 2, 4, 128, 128, 128
ROWS = N * C
ROW_ELEMS = D * H * W
SLAB = H * W

D_TC = 112
D_SC = D - D_TC

BLK_D = 32
ND = D_TC // BLK_D if D_TC else 0


def _tc_kernel(inp_ref, tgt_ref, out_pt, out_den):
    n = pl.program_id(0)
    c = pl.program_id(1)
    d = pl.program_id(2)
    row = n * C + c

    @pl.when(jnp.logical_and(row == 0, d == 0))
    def _init():
        out_pt[...] = jnp.zeros_like(out_pt)
        out_den[...] = jnp.zeros_like(out_den)

    x = inp_ref[0, 0]
    t = tgt_ref[0, 0]
    p = jax.nn.sigmoid(x)
    s_pt = jnp.sum(p * t)
    s_den = jnp.sum(p * p + t)

    row_mask = jax.lax.broadcasted_iota(jnp.int32, (ROWS, 128), 0) == row
    out_pt[...] += jnp.where(row_mask, s_pt, 0.0)
    out_den[...] += jnp.where(row_mask, s_den, 0.0)


def _tc_partials(input, target):
    return pl.pallas_call(
        _tc_kernel,
        grid=(N, C, ND),
        in_specs=[
            pl.BlockSpec((1, 1, BLK_D, H, W), lambda n, c, d: (n, c, d, 0, 0)),
            pl.BlockSpec((1, 1, BLK_D, H, W), lambda n, c, d: (n, c, d, 0, 0)),
        ],
        out_specs=[
            pl.BlockSpec((ROWS, 128), lambda n, c, d: (0, 0)),
            pl.BlockSpec((ROWS, 128), lambda n, c, d: (0, 0)),
        ],
        out_shape=[
            jax.ShapeDtypeStruct((ROWS, 128), jnp.float32),
            jax.ShapeDtypeStruct((ROWS, 128), jnp.float32),
        ],
    )(input, target)


CHUNK = 16384
NW = 32
SC_ROW = D_SC * SLAB
PER_W = SC_ROW // 4
NCH = PER_W // CHUNK if D_SC else 0

_sc_mesh = plsc.VectorSubcoreMesh(core_axis_name="c", subcore_axis_name="s")


@functools.partial(
    pl.kernel,
    out_type=jax.ShapeDtypeStruct((NW, 2, 16), jnp.float32),
    mesh=_sc_mesh,
    scratch_types=[
        pltpu.VMEM((CHUNK,), jnp.float32),
        pltpu.VMEM((CHUNK,), jnp.float32),
        pltpu.VMEM((CHUNK,), jnp.float32),
        pltpu.VMEM((CHUNK,), jnp.float32),
        pltpu.VMEM((2, 16), jnp.float32),
        pltpu.SemaphoreType.DMA,
        pltpu.SemaphoreType.DMA,
        pltpu.SemaphoreType.DMA,
        pltpu.SemaphoreType.DMA,
    ],
)
def _sc_partials(inp_hbm, tgt_hbm, out_hbm,
                 xbuf0, xbuf1, tbuf0, tbuf1, res,
                 semx0, semx1, semt0, semt1):
    w = lax.axis_index("s") * 2 + lax.axis_index("c")
    row = w // 4
    q = w % 4
    base = row * ROW_ELEMS + D_TC * SLAB + q * PER_W

    def start(g, xbuf, tbuf, semx, semt):
        off = pl.multiple_of(base + g * CHUNK, CHUNK)
        pltpu.async_copy(inp_hbm.at[pl.ds(off, CHUNK)], xbuf, semx)
        pltpu.async_copy(tgt_hbm.at[pl.ds(off, CHUNK)], tbuf, semt)

    def wait(xbuf, tbuf, semx, semt):
        pltpu.make_async_copy(inp_hbm.at[pl.ds(0, CHUNK)], xbuf, semx).wait()
        pltpu.make_async_copy(tgt_hbm.at[pl.ds(0, CHUNK)], tbuf, semt).wait()

    def compute(xbuf, tbuf, accs):
        def inner(k, accs2):
            o = pl.multiple_of(k * 64, 64)
            out = []
            for j in range(4):
                a_pt, a_den = accs2[2 * j], accs2[2 * j + 1]
                x = xbuf[pl.ds(o + j * 16, 16)]
                t = tbuf[pl.ds(o + j * 16, 16)]
                p = 1.0 / (1.0 + jnp.exp(-x))
                out.append(a_pt + p * t)
                out.append(a_den + (p * p + t))
            return tuple(out)

        return lax.fori_loop(0, CHUNK // 64, inner, accs, unroll=4)

    start(0, xbuf0, tbuf0, semx0, semt0)
    z = jnp.zeros((16,), jnp.float32)
    accs = (z,) * 8

    def two_chunks(gp, accs):
        g0 = gp * 2
        start(g0 + 1, xbuf1, tbuf1, semx1, semt1)
        wait(xbuf0, tbuf0, semx0, semt0)
        accs = compute(xbuf0, tbuf0, accs)

        @pl.when(g0 + 2 < NCH)
        def _():
            start(g0 + 2, xbuf0, tbuf0, semx0, semt0)

        wait(xbuf1, tbuf1, semx1, semt1)
        return compute(xbuf1, tbuf1, accs)

    accs = lax.fori_loop(0, NCH // 2, two_chunks, accs)
    a_pt = (accs[0] + accs[2]) + (accs[4] + accs[6])
    a_den = (accs[1] + accs[3]) + (accs[5] + accs[7])
    res[0, :] = a_pt
    res[1, :] = a_den
    pltpu.sync_copy(res, out_hbm.at[w])


def kernel(input, target):
    s_pt = jnp.zeros((ROWS,), jnp.float32)
    s_den = jnp.zeros((ROWS,), jnp.float32)

    if D_TC:
        tc_pt, tc_den = _tc_partials(input, target)
        s_pt = s_pt + tc_pt[:, 0]
        s_den = s_den + tc_den[:, 0]

    if D_SC:
        part = _sc_partials(input.reshape(-1), target.reshape(-1))
        s_pt = s_pt + part[:, 0, :].reshape(ROWS, 4 * 16).sum(-1)
        s_den = s_den + part[:, 1, :].reshape(ROWS, 4 * 16).sum(-1)

    intersect = s_pt[0:C] + s_pt[C:ROWS]
    denom = s_den[0:C] + s_den[C:ROWS]
    dice = 2.0 * intersect / jnp.maximum(denom, EPS)
    loss = 1.0 - jnp.mean(dice)
    return (loss, dice)


# speedup vs baseline: 4.0335x; 1.1376x over previous
import functools
import jax
import jax.numpy as jnp
from jax import lax
from jax.experimental import pallas as pl
from jax.experimental.pallas import tpu as pltpu
from jax.experimental.pallas import tpu_sc as plsc

EPS = 1e-6

N, C, D, H, W = 2, 4, 128, 128, 128
ROWS = N * C
ROW_ELEMS = D * H * W
SLAB = H * W

D_TC = 128
D_SC = D - D_TC

BLK_D = 32
ND = D_TC // BLK_D if D_TC else 0
assert ND * BLK_D == D_TC, "TC block split must cover [0, D_TC) exactly"


def _tc_kernel(inp_ref, tgt_ref, out_pt, out_den):
    n = pl.program_id(0)
    c = pl.program_id(1)
    d = pl.program_id(2)
    row = n * C + c

    @pl.when(jnp.logical_and(row == 0, d == 0))
    def _init():
        out_pt[...] = jnp.zeros_like(out_pt)
        out_den[...] = jnp.zeros_like(out_den)

    x = inp_ref[0, 0]
    t = tgt_ref[0, 0]
    p = jax.nn.sigmoid(x)
    s_pt = jnp.sum(p * t)
    s_den = jnp.sum(p * p + t)

    row_mask = jax.lax.broadcasted_iota(jnp.int32, (ROWS, 128), 0) == row
    out_pt[...] += jnp.where(row_mask, s_pt, 0.0)
    out_den[...] += jnp.where(row_mask, s_den, 0.0)


def _tc_partials(input, target):
    return pl.pallas_call(
        _tc_kernel,
        grid=(N, C, ND),
        in_specs=[
            pl.BlockSpec((1, 1, BLK_D, H, W), lambda n, c, d: (n, c, d, 0, 0)),
            pl.BlockSpec((1, 1, BLK_D, H, W), lambda n, c, d: (n, c, d, 0, 0)),
        ],
        out_specs=[
            pl.BlockSpec((ROWS, 128), lambda n, c, d: (0, 0)),
            pl.BlockSpec((ROWS, 128), lambda n, c, d: (0, 0)),
        ],
        out_shape=[
            jax.ShapeDtypeStruct((ROWS, 128), jnp.float32),
            jax.ShapeDtypeStruct((ROWS, 128), jnp.float32),
        ],
    )(input, target)


CHUNK = 16384
NW = 32
SC_ROW = D_SC * SLAB
PER_W = SC_ROW // 4
NCH = PER_W // CHUNK if D_SC else 0

_sc_mesh = plsc.VectorSubcoreMesh(core_axis_name="c", subcore_axis_name="s")


@functools.partial(
    pl.kernel,
    out_type=jax.ShapeDtypeStruct((NW, 2, 16), jnp.float32),
    mesh=_sc_mesh,
    scratch_types=[
        pltpu.VMEM((CHUNK,), jnp.float32),
        pltpu.VMEM((CHUNK,), jnp.float32),
        pltpu.VMEM((CHUNK,), jnp.float32),
        pltpu.VMEM((CHUNK,), jnp.float32),
        pltpu.VMEM((2, 16), jnp.float32),
        pltpu.SemaphoreType.DMA,
        pltpu.SemaphoreType.DMA,
        pltpu.SemaphoreType.DMA,
        pltpu.SemaphoreType.DMA,
    ],
)
def _sc_partials(inp_hbm, tgt_hbm, out_hbm,
                 xbuf0, xbuf1, tbuf0, tbuf1, res,
                 semx0, semx1, semt0, semt1):
    w = lax.axis_index("s") * 2 + lax.axis_index("c")
    row = w // 4
    q = w % 4
    base = row * ROW_ELEMS + D_TC * SLAB + q * PER_W

    def start(g, xbuf, tbuf, semx, semt):
        off = pl.multiple_of(base + g * CHUNK, CHUNK)
        pltpu.async_copy(inp_hbm.at[pl.ds(off, CHUNK)], xbuf, semx)
        pltpu.async_copy(tgt_hbm.at[pl.ds(off, CHUNK)], tbuf, semt)

    def wait(xbuf, tbuf, semx, semt):
        pltpu.make_async_copy(inp_hbm.at[pl.ds(0, CHUNK)], xbuf, semx).wait()
        pltpu.make_async_copy(tgt_hbm.at[pl.ds(0, CHUNK)], tbuf, semt).wait()

    def compute(xbuf, tbuf, accs):
        def inner(k, accs2):
            o = pl.multiple_of(k * 64, 64)
            out = []
            for j in range(4):
                a_pt, a_den = accs2[2 * j], accs2[2 * j + 1]
                x = xbuf[pl.ds(o + j * 16, 16)]
                t = tbuf[pl.ds(o + j * 16, 16)]
                p = 1.0 / (1.0 + jnp.exp(-x))
                out.append(a_pt + p * t)
                out.append(a_den + (p * p + t))
            return tuple(out)

        return lax.fori_loop(0, CHUNK // 64, inner, accs, unroll=4)

    start(0, xbuf0, tbuf0, semx0, semt0)
    z = jnp.zeros((16,), jnp.float32)
    accs = (z,) * 8

    def two_chunks(gp, accs):
        g0 = gp * 2
        start(g0 + 1, xbuf1, tbuf1, semx1, semt1)
        wait(xbuf0, tbuf0, semx0, semt0)
        accs = compute(xbuf0, tbuf0, accs)

        @pl.when(g0 + 2 < NCH)
        def _():
            start(g0 + 2, xbuf0, tbuf0, semx0, semt0)

        wait(xbuf1, tbuf1, semx1, semt1)
        return compute(xbuf1, tbuf1, accs)

    accs = lax.fori_loop(0, NCH // 2, two_chunks, accs)
    a_pt = (accs[0] + accs[2]) + (accs[4] + accs[6])
    a_den = (accs[1] + accs[3]) + (accs[5] + accs[7])
    res[0, :] = a_pt
    res[1, :] = a_den
    pltpu.sync_copy(res, out_hbm.at[w])


def kernel(input, target):
    s_pt = jnp.zeros((ROWS,), jnp.float32)
    s_den = jnp.zeros((ROWS,), jnp.float32)

    if D_SC:
        part = _sc_partials(input.reshape(-1), target.reshape(-1))
        s_pt = s_pt + part[:, 0, :].reshape(ROWS, 4 * 16).sum(-1)
        s_den = s_den + part[:, 1, :].reshape(ROWS, 4 * 16).sum(-1)

    if D_TC:
        tc_pt, tc_den = _tc_partials(input, target)
        s_pt = s_pt + tc_pt[:, 0]
        s_den = s_den + tc_den[:, 0]

    intersect = s_pt[0:C] + s_pt[C:ROWS]
    denom = s_den[0:C] + s_den[C:ROWS]
    dice = 2.0 * intersect / jnp.maximum(denom, EPS)
    loss = 1.0 - jnp.mean(dice)
    return (loss, dice)


# speedup vs baseline: 4.6285x; 1.1475x over previous
import functools
import jax
import jax.numpy as jnp
from jax import lax
from jax.experimental import pallas as pl
from jax.experimental.pallas import tpu as pltpu
from jax.experimental.pallas import tpu_sc as plsc

EPS = 1e-6

N, C, D, H, W = 2, 4, 128, 128, 128
ROWS = N * C
ROW_ELEMS = D * H * W
SLAB = H * W

D_TC = 128
D_SC = D - D_TC

BLK_D = 64
ND = D_TC // BLK_D if D_TC else 0
assert ND * BLK_D == D_TC, "TC block split must cover [0, D_TC) exactly"


def _tc_kernel(inp_ref, tgt_ref, out_pt, out_den):
    n = pl.program_id(0)
    c = pl.program_id(1)
    d = pl.program_id(2)
    row = n * C + c

    @pl.when(jnp.logical_and(row == 0, d == 0))
    def _init():
        out_pt[...] = jnp.zeros_like(out_pt)
        out_den[...] = jnp.zeros_like(out_den)

    x = inp_ref[0, 0]
    t = tgt_ref[0, 0]
    p = jax.nn.sigmoid(x)
    s_pt = jnp.sum(p * t)
    s_den = jnp.sum(p * p + t)

    row_mask = jax.lax.broadcasted_iota(jnp.int32, (ROWS, 128), 0) == row
    out_pt[...] += jnp.where(row_mask, s_pt, 0.0)
    out_den[...] += jnp.where(row_mask, s_den, 0.0)


def _tc_partials(input, target):
    return pl.pallas_call(
        _tc_kernel,
        grid=(N, C, ND),
        in_specs=[
            pl.BlockSpec((1, 1, BLK_D, H, W), lambda n, c, d: (n, c, d, 0, 0)),
            pl.BlockSpec((1, 1, BLK_D, H, W), lambda n, c, d: (n, c, d, 0, 0)),
        ],
        out_specs=[
            pl.BlockSpec((ROWS, 128), lambda n, c, d: (0, 0)),
            pl.BlockSpec((ROWS, 128), lambda n, c, d: (0, 0)),
        ],
        out_shape=[
            jax.ShapeDtypeStruct((ROWS, 128), jnp.float32),
            jax.ShapeDtypeStruct((ROWS, 128), jnp.float32),
        ],
    )(input, target)


CHUNK = 16384
NW = 32
SC_ROW = D_SC * SLAB
PER_W = SC_ROW // 4
NCH = PER_W // CHUNK if D_SC else 0

_sc_mesh = plsc.VectorSubcoreMesh(core_axis_name="c", subcore_axis_name="s")


@functools.partial(
    pl.kernel,
    out_type=jax.ShapeDtypeStruct((NW, 2, 16), jnp.float32),
    mesh=_sc_mesh,
    scratch_types=[
        pltpu.VMEM((CHUNK,), jnp.float32),
        pltpu.VMEM((CHUNK,), jnp.float32),
        pltpu.VMEM((CHUNK,), jnp.float32),
        pltpu.VMEM((CHUNK,), jnp.float32),
        pltpu.VMEM((2, 16), jnp.float32),
        pltpu.SemaphoreType.DMA,
        pltpu.SemaphoreType.DMA,
        pltpu.SemaphoreType.DMA,
        pltpu.SemaphoreType.DMA,
    ],
)
def _sc_partials(inp_hbm, tgt_hbm, out_hbm,
                 xbuf0, xbuf1, tbuf0, tbuf1, res,
                 semx0, semx1, semt0, semt1):
    w = lax.axis_index("s") * 2 + lax.axis_index("c")
    row = w // 4
    q = w % 4
    base = row * ROW_ELEMS + D_TC * SLAB + q * PER_W

    def start(g, xbuf, tbuf, semx, semt):
        off = pl.multiple_of(base + g * CHUNK, CHUNK)
        pltpu.async_copy(inp_hbm.at[pl.ds(off, CHUNK)], xbuf, semx)
        pltpu.async_copy(tgt_hbm.at[pl.ds(off, CHUNK)], tbuf, semt)

    def wait(xbuf, tbuf, semx, semt):
        pltpu.make_async_copy(inp_hbm.at[pl.ds(0, CHUNK)], xbuf, semx).wait()
        pltpu.make_async_copy(tgt_hbm.at[pl.ds(0, CHUNK)], tbuf, semt).wait()

    def compute(xbuf, tbuf, accs):
        def inner(k, accs2):
            o = pl.multiple_of(k * 64, 64)
            out = []
            for j in range(4):
                a_pt, a_den = accs2[2 * j], accs2[2 * j + 1]
                x = xbuf[pl.ds(o + j * 16, 16)]
                t = tbuf[pl.ds(o + j * 16, 16)]
                p = 1.0 / (1.0 + jnp.exp(-x))
                out.append(a_pt + p * t)
                out.append(a_den + (p * p + t))
            return tuple(out)

        return lax.fori_loop(0, CHUNK // 64, inner, accs, unroll=4)

    start(0, xbuf0, tbuf0, semx0, semt0)
    z = jnp.zeros((16,), jnp.float32)
    accs = (z,) * 8

    def two_chunks(gp, accs):
        g0 = gp * 2
        start(g0 + 1, xbuf1, tbuf1, semx1, semt1)
        wait(xbuf0, tbuf0, semx0, semt0)
        accs = compute(xbuf0, tbuf0, accs)

        @pl.when(g0 + 2 < NCH)
        def _():
            start(g0 + 2, xbuf0, tbuf0, semx0, semt0)

        wait(xbuf1, tbuf1, semx1, semt1)
        return compute(xbuf1, tbuf1, accs)

    accs = lax.fori_loop(0, NCH // 2, two_chunks, accs)
    a_pt = (accs[0] + accs[2]) + (accs[4] + accs[6])
    a_den = (accs[1] + accs[3]) + (accs[5] + accs[7])
    res[0, :] = a_pt
    res[1, :] = a_den
    pltpu.sync_copy(res, out_hbm.at[w])


def kernel(input, target):
    s_pt = jnp.zeros((ROWS,), jnp.float32)
    s_den = jnp.zeros((ROWS,), jnp.float32)

    if D_SC:
        part = _sc_partials(input.reshape(-1), target.reshape(-1))
        s_pt = s_pt + part[:, 0, :].reshape(ROWS, 4 * 16).sum(-1)
        s_den = s_den + part[:, 1, :].reshape(ROWS, 4 * 16).sum(-1)

    if D_TC:
        tc_pt, tc_den = _tc_partials(input, target)
        s_pt = s_pt + tc_pt[:, 0]
        s_den = s_den + tc_den[:, 0]

    intersect = s_pt[0:C] + s_pt[C:ROWS]
    denom = s_den[0:C] + s_den[C:ROWS]
    dice = 2.0 * intersect / jnp.maximum(denom, EPS)
    loss = 1.0 - jnp.mean(dice)
    return (loss, dice)


# speedup vs baseline: 4.8871x; 1.0559x over previous
import functools
import jax
import jax.numpy as jnp
from jax import lax
from jax.experimental import pallas as pl
from jax.experimental.pallas import tpu as pltpu
from jax.experimental.pallas import tpu_sc as plsc

EPS = 1e-6

N, C, D, H, W = 2, 4, 128, 128, 128
ROWS = N * C
ROW_ELEMS = D * H * W
SLAB = H * W

D_TC = 128
D_SC = D - D_TC

BLK_D = 128
ND = D_TC // BLK_D if D_TC else 0
assert ND * BLK_D == D_TC, "TC block split must cover [0, D_TC) exactly"


def _tc_kernel(inp_ref, tgt_ref, out_pt, out_den):
    n = pl.program_id(0)
    c = pl.program_id(1)
    d = pl.program_id(2)
    row = n * C + c

    @pl.when(jnp.logical_and(row == 0, d == 0))
    def _init():
        out_pt[...] = jnp.zeros_like(out_pt)
        out_den[...] = jnp.zeros_like(out_den)

    x = inp_ref[0, 0]
    t = tgt_ref[0, 0]
    p = jax.nn.sigmoid(x)
    s_pt = jnp.sum(p * t)
    s_den = jnp.sum(p * p + t)

    row_mask = jax.lax.broadcasted_iota(jnp.int32, (ROWS, 128), 0) == row
    out_pt[...] += jnp.where(row_mask, s_pt, 0.0)
    out_den[...] += jnp.where(row_mask, s_den, 0.0)


def _tc_partials(input, target):
    return pl.pallas_call(
        _tc_kernel,
        grid=(N, C, ND),
        in_specs=[
            pl.BlockSpec((1, 1, BLK_D, H, W), lambda n, c, d: (n, c, d, 0, 0)),
            pl.BlockSpec((1, 1, BLK_D, H, W), lambda n, c, d: (n, c, d, 0, 0)),
        ],
        out_specs=[
            pl.BlockSpec((ROWS, 128), lambda n, c, d: (0, 0)),
            pl.BlockSpec((ROWS, 128), lambda n, c, d: (0, 0)),
        ],
        out_shape=[
            jax.ShapeDtypeStruct((ROWS, 128), jnp.float32),
            jax.ShapeDtypeStruct((ROWS, 128), jnp.float32),
        ],
    )(input, target)


CHUNK = 16384
NW = 32
SC_ROW = D_SC * SLAB
PER_W = SC_ROW // 4
NCH = PER_W // CHUNK if D_SC else 0

_sc_mesh = plsc.VectorSubcoreMesh(core_axis_name="c", subcore_axis_name="s")


@functools.partial(
    pl.kernel,
    out_type=jax.ShapeDtypeStruct((NW, 2, 16), jnp.float32),
    mesh=_sc_mesh,
    scratch_types=[
        pltpu.VMEM((CHUNK,), jnp.float32),
        pltpu.VMEM((CHUNK,), jnp.float32),
        pltpu.VMEM((CHUNK,), jnp.float32),
        pltpu.VMEM((CHUNK,), jnp.float32),
        pltpu.VMEM((2, 16), jnp.float32),
        pltpu.SemaphoreType.DMA,
        pltpu.SemaphoreType.DMA,
        pltpu.SemaphoreType.DMA,
        pltpu.SemaphoreType.DMA,
    ],
)
def _sc_partials(inp_hbm, tgt_hbm, out_hbm,
                 xbuf0, xbuf1, tbuf0, tbuf1, res,
                 semx0, semx1, semt0, semt1):
    w = lax.axis_index("s") * 2 + lax.axis_index("c")
    row = w // 4
    q = w % 4
    base = row * ROW_ELEMS + D_TC * SLAB + q * PER_W

    def start(g, xbuf, tbuf, semx, semt):
        off = pl.multiple_of(base + g * CHUNK, CHUNK)
        pltpu.async_copy(inp_hbm.at[pl.ds(off, CHUNK)], xbuf, semx)
        pltpu.async_copy(tgt_hbm.at[pl.ds(off, CHUNK)], tbuf, semt)

    def wait(xbuf, tbuf, semx, semt):
        pltpu.make_async_copy(inp_hbm.at[pl.ds(0, CHUNK)], xbuf, semx).wait()
        pltpu.make_async_copy(tgt_hbm.at[pl.ds(0, CHUNK)], tbuf, semt).wait()

    def compute(xbuf, tbuf, accs):
        def inner(k, accs2):
            o = pl.multiple_of(k * 64, 64)
            out = []
            for j in range(4):
                a_pt, a_den = accs2[2 * j], accs2[2 * j + 1]
                x = xbuf[pl.ds(o + j * 16, 16)]
                t = tbuf[pl.ds(o + j * 16, 16)]
                p = 1.0 / (1.0 + jnp.exp(-x))
                out.append(a_pt + p * t)
                out.append(a_den + (p * p + t))
            return tuple(out)

        return lax.fori_loop(0, CHUNK // 64, inner, accs, unroll=4)

    start(0, xbuf0, tbuf0, semx0, semt0)
    z = jnp.zeros((16,), jnp.float32)
    accs = (z,) * 8

    def two_chunks(gp, accs):
        g0 = gp * 2
        start(g0 + 1, xbuf1, tbuf1, semx1, semt1)
        wait(xbuf0, tbuf0, semx0, semt0)
        accs = compute(xbuf0, tbuf0, accs)

        @pl.when(g0 + 2 < NCH)
        def _():
            start(g0 + 2, xbuf0, tbuf0, semx0, semt0)

        wait(xbuf1, tbuf1, semx1, semt1)
        return compute(xbuf1, tbuf1, accs)

    accs = lax.fori_loop(0, NCH // 2, two_chunks, accs)
    a_pt = (accs[0] + accs[2]) + (accs[4] + accs[6])
    a_den = (accs[1] + accs[3]) + (accs[5] + accs[7])
    res[0, :] = a_pt
    res[1, :] = a_den
    pltpu.sync_copy(res, out_hbm.at[w])


def kernel(input, target):
    s_pt = jnp.zeros((ROWS,), jnp.float32)
    s_den = jnp.zeros((ROWS,), jnp.float32)

    if D_SC:
        part = _sc_partials(input.reshape(-1), target.reshape(-1))
        s_pt = s_pt + part[:, 0, :].reshape(ROWS, 4 * 16).sum(-1)
        s_den = s_den + part[:, 1, :].reshape(ROWS, 4 * 16).sum(-1)

    if D_TC:
        tc_pt, tc_den = _tc_partials(input, target)
        s_pt = s_pt + tc_pt[:, 0]
        s_den = s_den + tc_den[:, 0]

    intersect = s_pt[0:C] + s_pt[C:ROWS]
    denom = s_den[0:C] + s_den[C:ROWS]
    dice = 2.0 * intersect / jnp.maximum(denom, EPS)
    loss = 1.0 - jnp.mean(dice)
    return (loss, dice)
